# baseline (speedup 1.0000x reference)
import functools

import jax
import jax.numpy as jnp
from jax import lax
from jax.experimental import pallas as pl
from jax.experimental.pallas import tpu as pltpu
from jax.experimental.pallas import tpu_sc as plsc

BATCH = 16384
D = 16
L = 16
G = 2


@functools.lru_cache(maxsize=1)
def _build():
    info = plsc.get_sparse_core_info()
    nc, ns = info.num_cores, info.num_subcores
    nw = nc * ns
    bpw = BATCH // nw
    nbody = bpw // (G * L)
    mesh = plsc.VectorSubcoreMesh(core_axis_name="c", subcore_axis_name="s")

    @functools.partial(
        pl.kernel,
        mesh=mesh,
        compiler_params=pltpu.CompilerParams(
            needs_layout_passes=False, use_tc_tiling_on_sc=True),
        out_type=jax.ShapeDtypeStruct((BATCH,), jnp.float32),
        scratch_types=[
            pltpu.VMEM((bpw,), jnp.int32),
            pltpu.VMEM((bpw,), jnp.int32),
            pltpu.VMEM((G * L, D), jnp.float32),
            pltpu.VMEM((G * L, D), jnp.float32),
            pltpu.VMEM((bpw,), jnp.float32),
            pltpu.SemaphoreType.DMA,
            pltpu.SemaphoreType.DMA,
        ],
    )
    def k(idx_i_hbm, idx_j_hbm, table_hbm, out_hbm,
          idxi_v, idxj_v, rows_i, rows_j, out_v, sem_i, sem_j):
        wid = lax.axis_index("s") * nc + lax.axis_index("c")
        base = wid * bpw
        pltpu.sync_copy(idx_i_hbm.at[pl.ds(base, bpw)], idxi_v)
        pltpu.sync_copy(idx_j_hbm.at[pl.ds(base, bpw)], idxj_v)

        @pl.loop(0, nbody)
        def body(it):
            b0 = it * (G * L)
            copies = []
            for g in range(G):
                s = pl.ds(b0 + g * L, L)
                tiv = idxi_v[s]
                tjv = idxj_v[s]
                for t in range(L):
                    copies.append(pltpu.async_copy(
                        table_hbm.at[tiv[t]], rows_i.at[g * L + t], sem_i))
                    copies.append(pltpu.async_copy(
                        table_hbm.at[tjv[t]], rows_j.at[g * L + t], sem_j))
            for cp in copies:
                cp.wait()
            for g in range(G):
                s = pl.ds(b0 + g * L, L)
                bvec = g * L + lax.iota(jnp.int32, L)
                acc = jnp.zeros((L,), jnp.float32)
                for kk in range(D):
                    col = jnp.full((L,), kk, jnp.int32)
                    a = plsc.load_gather(rows_i, [bvec, col])
                    b = plsc.load_gather(rows_j, [bvec, col])
                    acc = acc + a * b
                out_v[s] = acc

        pltpu.sync_copy(out_v, out_hbm.at[pl.ds(base, bpw)])

    return k


def kernel(user_index_i, user_index_j, user_embedding):
    k = _build()
    return k(user_index_i.astype(jnp.int32),
             user_index_j.astype(jnp.int32),
             user_embedding)
